# baseline (speedup 1.0000x reference)
import functools

import jax
import jax.numpy as jnp
from jax import lax
from jax.experimental import pallas as pl
from jax.experimental.pallas import tpu as pltpu
from jax.experimental.pallas import tpu_sc as plsc

B = 8
T = 8192
V = 100000
D = 64
NC = 8
NS = 8
H = 512
OUT = 128

ROWS = T * NS
NUM_WORKERS = 32
ROWS_PER_W = ROWS // NUM_WORKERS
CHUNK = 128
N_CHUNKS = ROWS_PER_W // CHUNK
CTX_N = B * NC

TM = 512
GRID = T // TM


def _sc_gather_body(seq_pair_tab, idx_all, ctx_pair_tab, cidx_in,
                    seq_out, ctx_slots,
                    idx_v, buf_a, buf_b, cidx, cbuf, sem):
    c = lax.axis_index("c")
    s = lax.axis_index("s")
    wid = s * 2 + c
    base = wid * ROWS_PER_W
    pltpu.sync_copy(idx_all.at[pl.ds(wid * N_CHUNKS, N_CHUNKS)], idx_v)

    bufs = (buf_a, buf_b)

    def start(k, buf):
        return pltpu.async_copy(seq_pair_tab.at[idx_v.at[k]], buf, sem)

    cps = [start(0, bufs[0]), start(1, bufs[1])]
    for k in range(N_CHUNKS):
        cps[k].wait()
        pltpu.sync_copy(bufs[k % 2],
                        seq_out.at[pl.ds(base + k * CHUNK, CHUNK)])
        if k + 2 < N_CHUNKS:
            cps.append(start(k + 2, bufs[k % 2]))

    @pl.when(wid == 0)
    def _():
        pltpu.sync_copy(cidx_in, cidx)
        pltpu.async_copy(ctx_pair_tab.at[cidx.at[0]], cbuf, sem).wait()
        pltpu.sync_copy(cbuf, ctx_slots)


@functools.cache
def _sc_gather():
    return pl.kernel(
        _sc_gather_body,
        out_type=[
            jax.ShapeDtypeStruct((ROWS, 2 * D), jnp.float32),
            jax.ShapeDtypeStruct((CTX_N, 2 * D), jnp.float32),
        ],
        mesh=plsc.VectorSubcoreMesh(core_axis_name="c", subcore_axis_name="s"),
        compiler_params=pltpu.CompilerParams(use_tc_tiling_on_sc=True),
        scratch_types=[
            pltpu.VMEM((N_CHUNKS, CHUNK), jnp.int32),
            pltpu.VMEM((CHUNK, 2 * D), jnp.float32),
            pltpu.VMEM((CHUNK, 2 * D), jnp.float32),
            pltpu.VMEM((1, CTX_N), jnp.int32),
            pltpu.VMEM((CTX_N, 2 * D), jnp.float32),
            pltpu.SemaphoreType.DMA,
        ],
    )


def _tc_body(x_ref, ids_ref, slots_ref, cpar_ref, rid_ref,
             wc1, bc1, wc2, bc2,
             w01, b01, w02, b02, w11, b11, w12, b12,
             ce_ref, ae_ref, sc_ref):
    @pl.when(pl.program_id(0) == 0)
    def _():
        slots = slots_ref[...]
        valid = jnp.where(cpar_ref[...] == 1,
                          slots[:, D:2 * D], slots[:, 0:D])
        ctx_x = jnp.concatenate(
            [valid[f * B:(f + 1) * B] for f in range(NC)], axis=1)
        hc = jnp.maximum(
            jnp.dot(ctx_x, wc1[...],
                    preferred_element_type=jnp.float32) + bc1[...], 0.0)
        ce_ref[...] = jnp.dot(hc, wc2[...],
                              preferred_element_type=jnp.float32) + bc2[...]

    pieces = []
    for f in range(NS):
        raw = x_ref[f]
        par = (ids_ref[f][:, None] & 1) == 1
        pieces.append(jnp.where(par, raw[:, D:2 * D], raw[:, 0:D]))
    x = jnp.concatenate(pieces, axis=1)

    h0 = jnp.maximum(
        jnp.dot(x, w01[...], preferred_element_type=jnp.float32) + b01[...],
        0.0)
    a0 = jnp.dot(h0, w02[...], preferred_element_type=jnp.float32) + b02[...]
    h1 = jnp.maximum(
        jnp.dot(x, w11[...], preferred_element_type=jnp.float32) + b11[...],
        0.0)
    a1 = jnp.dot(h1, w12[...], preferred_element_type=jnp.float32) + b12[...]
    ae_ref[0] = a0
    ae_ref[1] = a1

    ce = ce_ref[...]
    onehot = (rid_ref[...] == lax.broadcasted_iota(jnp.int32, (TM, B), 1)
              ).astype(jnp.float32)
    aligned = jnp.dot(onehot, ce, preferred_element_type=jnp.float32)
    s0 = jnp.sum(aligned * a0, axis=-1, keepdims=True)
    s1 = jnp.sum(aligned * a1, axis=-1, keepdims=True)
    sc_ref[...] = jnp.concatenate([s0, s1], axis=1)


_tc_grid_spec = dict(
    grid=(GRID,),
    in_specs=[
        pl.BlockSpec((NS, TM, 128), lambda i: (0, i, 0)),
        pl.BlockSpec((NS, TM), lambda i: (0, i)),
        pl.BlockSpec((CTX_N, 2 * D), lambda i: (0, 0)),
        pl.BlockSpec((CTX_N, 1), lambda i: (0, 0)),
        pl.BlockSpec((TM, 1), lambda i: (i, 0)),
        pl.BlockSpec((NC * D, H), lambda i: (0, 0)),
        pl.BlockSpec((1, H), lambda i: (0, 0)),
        pl.BlockSpec((H, OUT), lambda i: (0, 0)),
        pl.BlockSpec((1, OUT), lambda i: (0, 0)),
        pl.BlockSpec((NS * D, H), lambda i: (0, 0)),
        pl.BlockSpec((1, H), lambda i: (0, 0)),
        pl.BlockSpec((H, OUT), lambda i: (0, 0)),
        pl.BlockSpec((1, OUT), lambda i: (0, 0)),
        pl.BlockSpec((NS * D, H), lambda i: (0, 0)),
        pl.BlockSpec((1, H), lambda i: (0, 0)),
        pl.BlockSpec((H, OUT), lambda i: (0, 0)),
        pl.BlockSpec((1, OUT), lambda i: (0, 0)),
    ],
    out_specs=[
        pl.BlockSpec((B, OUT), lambda i: (0, 0)),
        pl.BlockSpec((2, TM, OUT), lambda i: (0, i, 0)),
        pl.BlockSpec((TM, 2), lambda i: (i, 0)),
    ],
)


def kernel(context_ids, seq_ids, row_ids, ctx_table, seq_table,
           Wc1, bc1, Wc2, bc2,
           Wa0_1, ba0_1, Wa0_2, ba0_2,
           Wa1_1, ba1_1, Wa1_2, ba1_2):
    ids_t = seq_ids.astype(jnp.int32).T
    idx_all = (ids_t // 2).reshape(ROWS // CHUNK, CHUNK)

    cflat = context_ids.astype(jnp.int32).T.reshape(-1)
    cidx = (cflat // 2).reshape(1, CTX_N)
    cpar = (cflat % 2).reshape(CTX_N, 1)

    seq_pair_tab = seq_table.reshape(V // 2, 2 * D)
    ctx_pair_tab = ctx_table.reshape(V // 2, 2 * D)

    seq_rows, ctx_slots = _sc_gather()(seq_pair_tab, idx_all,
                                       ctx_pair_tab, cidx)
    x3d = seq_rows.reshape(NS, T, 2 * D)
    rid2d = row_ids.reshape(T, 1).astype(jnp.int32)

    ce, ae, scores = pl.pallas_call(
        _tc_body,
        out_shape=[
            jax.ShapeDtypeStruct((B, OUT), jnp.float32),
            jax.ShapeDtypeStruct((2, T, OUT), jnp.float32),
            jax.ShapeDtypeStruct((T, 2), jnp.float32),
        ],
        compiler_params=pltpu.CompilerParams(
            dimension_semantics=("arbitrary",)),
        **_tc_grid_spec,
    )(x3d, ids_t, ctx_slots, cpar, rid2d,
      Wc1, bc1.reshape(1, H), Wc2, bc2.reshape(1, OUT),
      Wa0_1, ba0_1.reshape(1, H), Wa0_2, ba0_2.reshape(1, OUT),
      Wa1_1, ba1_1.reshape(1, H), Wa1_2, ba1_2.reshape(1, OUT))
    return ce, ae, scores


# speedup vs baseline: 1.2474x; 1.2474x over previous
import functools

import jax
import jax.numpy as jnp
from jax import lax
from jax.experimental import pallas as pl
from jax.experimental.pallas import tpu as pltpu
from jax.experimental.pallas import tpu_sc as plsc

B = 8
T = 8192
V = 100000
D = 64
NC = 8
NS = 8
H = 512
OUT = 128

NUM_WORKERS = 32
DIMS_PER_W = D // NUM_WORKERS
NCC = T // 128
CTX_N = B * NC

TM = 512
GRID = T // TM


def _sc_body(seq_t, ids_flat, ctx_t, cids_hbm, x_out, ctx_out,
             rowbuf, idbuf, outbuf, cidbuf, coutbuf, sem, wsem):
    c = lax.axis_index("c")
    s = lax.axis_index("s")
    wid = s * 2 + c
    zeros = jnp.zeros((16,), jnp.int32)

    pltpu.sync_copy(cids_hbm, cidbuf)

    for rr in range(DIMS_PER_W):
        d = wid * DIMS_PER_W + rr
        pltpu.sync_copy(seq_t.at[d], rowbuf)
        for f in range(NS):
            pltpu.sync_copy(ids_flat.at[pl.ds(f * T, T)], idbuf)

            def gbody(i, carry):
                idx = idbuf[pl.ds(i * 16, 16)]
                outbuf[pl.ds(i * 16, 16)] = plsc.load_gather(rowbuf, [idx])
                return carry

            lax.fori_loop(0, T // 16, gbody, 0)
            r = f * D + d

            def fire(cc, carry):
                pltpu.async_copy(outbuf.at[pl.ds(cc * 128, 128)],
                                 x_out.at[pl.ds(cc * (NS * D * 128) + r * 128,
                                                128)], wsem)
                return carry

            def drain(cc, carry):
                pltpu.make_async_copy(
                    outbuf.at[pl.ds(cc * 128, 128)],
                    x_out.at[pl.ds(cc * (NS * D * 128) + r * 128, 128)],
                    wsem).wait()
                return carry

            lax.fori_loop(0, NCC, fire, 0)
            lax.fori_loop(0, NCC, drain, 0)

        pltpu.sync_copy(ctx_t.at[d], rowbuf)
        for k in range(CTX_N // 16):
            cidx = cidbuf[pl.ds(k * 16, 16)]
            coutbuf[pl.ds(k * 16, 16)] = plsc.load_gather(rowbuf, [cidx])
        pltpu.sync_copy(coutbuf, ctx_out.at[pl.ds(d * CTX_N, CTX_N)])


@functools.cache
def _sc_gather():
    return pl.kernel(
        _sc_body,
        out_type=[
            jax.ShapeDtypeStruct((NCC * NS * D * 128,), jnp.float32),
            jax.ShapeDtypeStruct((D * CTX_N,), jnp.float32),
        ],
        mesh=plsc.VectorSubcoreMesh(core_axis_name="c", subcore_axis_name="s"),
        compiler_params=pltpu.CompilerParams(use_tc_tiling_on_sc=True,
                                             needs_layout_passes=False),
        scratch_types=[
            pltpu.VMEM((V,), jnp.float32),
            pltpu.VMEM((T,), jnp.int32),
            pltpu.VMEM((T,), jnp.float32),
            pltpu.VMEM((CTX_N,), jnp.int32),
            pltpu.VMEM((CTX_N,), jnp.float32),
            pltpu.SemaphoreType.DMA,
            pltpu.SemaphoreType.DMA,
        ],
    )


def _tc_body(x_ref, ctx_ref, rid_ref,
             wc1, bc1, wc2, bc2,
             w01, b01, w02, b02, w11, b11, w12, b12,
             ce_ref, ae_ref, sc_ref):
    @pl.when(pl.program_id(0) == 0)
    def _():
        cin = ctx_ref[...]
        hc = bc1[...]
        for f in range(NC):
            hc = hc + lax.dot_general(
                cin[:, f * B:(f + 1) * B], wc1[f * D:(f + 1) * D, :],
                (((0,), (0,)), ((), ())),
                preferred_element_type=jnp.float32)
        hc = jnp.maximum(hc, 0.0)
        ce_ref[...] = jnp.dot(hc, wc2[...],
                              preferred_element_type=jnp.float32) + bc2[...]

    xt = jnp.concatenate([x_ref[j] for j in range(TM // 128)],
                         axis=1)
    h0 = jnp.maximum(
        lax.dot_general(w01[...], xt, (((0,), (0,)), ((), ())),
                        preferred_element_type=jnp.float32) + b01[...], 0.0)
    a0 = lax.dot_general(h0, w02[...], (((0,), (0,)), ((), ())),
                         preferred_element_type=jnp.float32) + b02[...]
    h1 = jnp.maximum(
        lax.dot_general(w11[...], xt, (((0,), (0,)), ((), ())),
                        preferred_element_type=jnp.float32) + b11[...], 0.0)
    a1 = lax.dot_general(h1, w12[...], (((0,), (0,)), ((), ())),
                         preferred_element_type=jnp.float32) + b12[...]
    ae_ref[0] = a0
    ae_ref[1] = a1

    ce = ce_ref[...]
    onehot = (rid_ref[...] == lax.broadcasted_iota(jnp.int32, (TM, B), 1)
              ).astype(jnp.float32)
    aligned = jnp.dot(onehot, ce, preferred_element_type=jnp.float32)
    s0 = jnp.sum(aligned * a0, axis=-1, keepdims=True)
    s1 = jnp.sum(aligned * a1, axis=-1, keepdims=True)
    sc_ref[...] = jnp.concatenate([s0, s1], axis=1)


_tc_grid_spec = dict(
    grid=(GRID,),
    in_specs=[
        pl.BlockSpec((TM // 128, NS * D, 128), lambda i: (i, 0, 0)),
        pl.BlockSpec((D, CTX_N), lambda i: (0, 0)),
        pl.BlockSpec((TM, 1), lambda i: (i, 0)),
        pl.BlockSpec((NC * D, H), lambda i: (0, 0)),
        pl.BlockSpec((1, H), lambda i: (0, 0)),
        pl.BlockSpec((H, OUT), lambda i: (0, 0)),
        pl.BlockSpec((1, OUT), lambda i: (0, 0)),
        pl.BlockSpec((NS * D, H), lambda i: (0, 0)),
        pl.BlockSpec((H, 1), lambda i: (0, 0)),
        pl.BlockSpec((H, OUT), lambda i: (0, 0)),
        pl.BlockSpec((1, OUT), lambda i: (0, 0)),
        pl.BlockSpec((NS * D, H), lambda i: (0, 0)),
        pl.BlockSpec((H, 1), lambda i: (0, 0)),
        pl.BlockSpec((H, OUT), lambda i: (0, 0)),
        pl.BlockSpec((1, OUT), lambda i: (0, 0)),
    ],
    out_specs=[
        pl.BlockSpec((B, OUT), lambda i: (0, 0)),
        pl.BlockSpec((2, TM, OUT), lambda i: (0, i, 0)),
        pl.BlockSpec((TM, 2), lambda i: (i, 0)),
    ],
)


def kernel(context_ids, seq_ids, row_ids, ctx_table, seq_table,
           Wc1, bc1, Wc2, bc2,
           Wa0_1, ba0_1, Wa0_2, ba0_2,
           Wa1_1, ba1_1, Wa1_2, ba1_2):
    seq_t = seq_table.T
    ctx_t = ctx_table.T
    ids_flat = seq_ids.astype(jnp.int32).T.reshape(-1)
    cids = context_ids.astype(jnp.int32).T.reshape(-1)

    x_flat, ctx_flat = _sc_gather()(seq_t, ids_flat, ctx_t, cids)
    x4 = x_flat.reshape(NCC, NS * D, 128)
    ctx_in = ctx_flat.reshape(D, CTX_N)
    rid2d = row_ids.reshape(T, 1).astype(jnp.int32)

    ce, ae, scores = pl.pallas_call(
        _tc_body,
        out_shape=[
            jax.ShapeDtypeStruct((B, OUT), jnp.float32),
            jax.ShapeDtypeStruct((2, T, OUT), jnp.float32),
            jax.ShapeDtypeStruct((T, 2), jnp.float32),
        ],
        compiler_params=pltpu.CompilerParams(
            dimension_semantics=("arbitrary",)),
        **_tc_grid_spec,
    )(x4, ctx_in, rid2d,
      Wc1, bc1.reshape(1, H), Wc2, bc2.reshape(1, OUT),
      Wa0_1, ba0_1.reshape(H, 1), Wa0_2, ba0_2.reshape(1, OUT),
      Wa1_1, ba1_1.reshape(H, 1), Wa1_2, ba1_2.reshape(1, OUT))
    return ce, ae, scores
